# baseline (speedup 1.0000x reference)
import functools

import jax
import jax.numpy as jnp
from jax import lax
from jax.experimental import pallas as pl
from jax.experimental.pallas import tpu as pltpu
from jax.experimental.pallas import tpu_sc as plsc

B, C, N_IN = 4, 128, 32768
N_OUT, K = 8192, 16
D = B * C
DW = D // 2
L = 16

NC, NS = 2, 16
NW = NC * NS
ROWS_PER_W = N_OUT // NW
G = 8
NBLK = ROWS_PER_W // G
NBUF = 3
NITER = NBLK // NBUF + (1 if NBLK % NBUF else 0)


def _sc_kernel_body(table_hbm, idx_hbm, out_hbm, idx_v,
                    rows_bufs, out_bufs, gsems, osems):
    wid = lax.axis_index("s") * NC + lax.axis_index("c")
    base = wid * ROWS_PER_W

    pltpu.sync_copy(idx_hbm.at[wid], idx_v)

    for b in range(NBUF):
        pltpu.async_copy(table_hbm.at[idx_v.at[b]], rows_bufs.at[b],
                         gsems.at[b])

    def iter_body(i, carry):
        for b in range(NBUF):
            g = i * NBUF + b

            @pl.when(g < NBLK)
            def _(b=b, g=g):
                rows_v = rows_bufs.at[b]
                out_v = out_bufs.at[b]
                pltpu.make_async_copy(table_hbm.at[idx_v.at[g]], rows_v,
                                      gsems.at[b]).wait()
                @pl.when(i > 0)
                def _():
                    pltpu.make_async_copy(
                        out_v, out_hbm.at[pl.ds(base, G)], osems.at[b]).wait()

                for r in range(G):
                    def dg_body(dg, c2, r=r):
                        off = dg * L
                        va, vb = [], []
                        for j in range(K):
                            w = rows_v[r * K + j, pl.ds(off, L)]
                            va.append(w << 16)
                            vb.append(w)
                        for vals in (va, vb):
                            while len(vals) > 1:
                                vals[:] = [
                                    jnp.maximum(vals[2 * m], vals[2 * m + 1])
                                    for m in range(len(vals) // 2)]
                        out_v[r, pl.ds(off, L)] = (
                            ((va[0] >> 16) & jnp.int32(0xFFFF))
                            | (vb[0] & jnp.int32(-65536)))
                        return c2
                    lax.fori_loop(0, DW // L, dg_body, 0, unroll=1)

                @pl.when(g + NBUF < NBLK)
                def _():
                    pltpu.async_copy(table_hbm.at[idx_v.at[g + NBUF]],
                                     rows_v, gsems.at[b])
                pltpu.async_copy(out_v, out_hbm.at[pl.ds(base + g * G, G)],
                                 osems.at[b])
        return carry

    lax.fori_loop(0, NITER, iter_body, 0)

    for b in range(NBUF):
        pltpu.make_async_copy(out_bufs.at[b], out_hbm.at[pl.ds(base, G)],
                              osems.at[b]).wait()


@jax.jit
def _max_pool_sc(table, idx_grouped):
    mesh = plsc.VectorSubcoreMesh(core_axis_name="c", subcore_axis_name="s")
    kfn = functools.partial(
        pl.kernel,
        mesh=mesh,
        out_type=jax.ShapeDtypeStruct((N_OUT, DW), jnp.int32),
        scratch_types=[
            pltpu.VMEM((NBLK, G * K), jnp.int32),
            pltpu.VMEM((NBUF, G * K, DW), jnp.int32),
            pltpu.VMEM((NBUF, G, DW), jnp.int32),
            pltpu.SemaphoreType.DMA((NBUF,)),
            pltpu.SemaphoreType.DMA((NBUF,)),
        ],
    )(_sc_kernel_body)
    return kfn(table, idx_grouped)


BN = 256
BP = 512


def _prep_body(x_ref, o_ref):
    bits = lax.bitcast_convert_type(x_ref[...], jnp.int32)
    rne = ((bits >> 16) & jnp.int32(1)) + jnp.int32(0x7FFF)
    bf = ((bits + rne) >> 16) & jnp.int32(0xFFFF)
    key = bf ^ (jnp.int32(0x7FFF) * ((bf >> 15) & jnp.int32(1)))
    words = key[:DW, :] | (key[DW:, :] << 16)
    o_ref[...] = words.T


def _post_body(w_ref, o_ref):
    w = w_ref[...]
    lo = w & jnp.int32(0xFFFF)
    hi = (w >> 16) & jnp.int32(0xFFFF)
    keys = jnp.concatenate([lo, hi], axis=1)
    ubf = keys ^ (jnp.int32(0x7FFF) * ((keys >> 15) & jnp.int32(1)))
    vals = lax.bitcast_convert_type(ubf << 16, jnp.float32)
    o_ref[...] = vals.T


@jax.jit
def _prep_tc(x2d):
    return pl.pallas_call(
        _prep_body,
        grid=(N_IN // BN,),
        in_specs=[pl.BlockSpec((D, BN), lambda i: (0, i))],
        out_specs=pl.BlockSpec((BN, DW), lambda i: (i, 0)),
        out_shape=jax.ShapeDtypeStruct((N_IN, DW), jnp.int32),
    )(x2d)


@jax.jit
def _post_tc(out_w):
    return pl.pallas_call(
        _post_body,
        grid=(N_OUT // BP,),
        in_specs=[pl.BlockSpec((BP, DW), lambda i: (i, 0))],
        out_specs=pl.BlockSpec((D, BP), lambda i: (0, i)),
        out_shape=jax.ShapeDtypeStruct((D, N_OUT), jnp.float32),
    )(out_w)


def kernel(x, idx):
    table = _prep_tc(x.reshape(D, N_IN))
    idx_grouped = idx.reshape(NW, NBLK, G * K)
    out_w = _max_pool_sc(table, idx_grouped)
    return _post_tc(out_w).reshape(B, C, N_OUT)


# speedup vs baseline: 1.2835x; 1.2835x over previous
import functools

import jax
import jax.numpy as jnp
from jax import lax
from jax.experimental import pallas as pl
from jax.experimental.pallas import tpu as pltpu
from jax.experimental.pallas import tpu_sc as plsc

B, C, N_IN = 4, 128, 32768
N_OUT, K = 8192, 16
D = B * C
DW = D // 2
L = 16

NC, NS = 2, 16
NW = NC * NS
ROWS_PER_W = N_OUT // NW
G = 8
NBLK = ROWS_PER_W // G
NBUF = 3
NITER = NBLK // NBUF + (1 if NBLK % NBUF else 0)


def _sc_kernel_body(table_hbm, idx_hbm, out_hbm, idx_v,
                    rows_bufs, out_bufs, gsems, osems):
    wid = lax.axis_index("s") * NC + lax.axis_index("c")
    base = wid * ROWS_PER_W

    pltpu.sync_copy(idx_hbm.at[wid], idx_v)

    for b in range(NBUF):
        pltpu.async_copy(table_hbm.at[idx_v.at[b]], rows_bufs.at[b],
                         gsems.at[b])

    def iter_body(i, carry):
        for b in range(NBUF):
            g = i * NBUF + b

            @pl.when(g < NBLK)
            def _(b=b, g=g):
                rows_v = rows_bufs.at[b]
                out_v = out_bufs.at[b]
                pltpu.make_async_copy(table_hbm.at[idx_v.at[g]], rows_v,
                                      gsems.at[b]).wait()
                @pl.when(i > 0)
                def _():
                    pltpu.make_async_copy(
                        out_v, out_hbm.at[pl.ds(base, G)], osems.at[b]).wait()

                for r in range(G):
                    def dg_body(dg, c2, r=r):
                        off = dg * L
                        va, vb = [], []
                        for j in range(K):
                            w = rows_v[r * K + j, pl.ds(off, L)]
                            va.append(w << 16)
                            vb.append(w)
                        for vals in (va, vb):
                            while len(vals) > 1:
                                vals[:] = [
                                    jnp.maximum(vals[2 * m], vals[2 * m + 1])
                                    for m in range(len(vals) // 2)]
                        out_v[r, pl.ds(off, L)] = (
                            ((va[0] >> 16) & jnp.int32(0xFFFF))
                            | (vb[0] & jnp.int32(-65536)))
                        return c2
                    lax.fori_loop(0, DW // L, dg_body, 0, unroll=1)

                @pl.when(g + NBUF < NBLK)
                def _():
                    pltpu.async_copy(table_hbm.at[idx_v.at[g + NBUF]],
                                     rows_v, gsems.at[b])
                pltpu.async_copy(out_v, out_hbm.at[pl.ds(base + g * G, G)],
                                 osems.at[b])
        return carry

    lax.fori_loop(0, NITER, iter_body, 0)

    for b in range(NBUF):
        pltpu.make_async_copy(out_bufs.at[b], out_hbm.at[pl.ds(base, G)],
                              osems.at[b]).wait()


@jax.jit
def _max_pool_sc(table, idx_grouped):
    mesh = plsc.VectorSubcoreMesh(core_axis_name="c", subcore_axis_name="s")
    kfn = functools.partial(
        pl.kernel,
        mesh=mesh,
        out_type=jax.ShapeDtypeStruct((N_OUT, DW), jnp.int32),
        scratch_types=[
            pltpu.VMEM((NBLK, G * K), jnp.int32),
            pltpu.VMEM((NBUF, G * K, DW), jnp.int32),
            pltpu.VMEM((NBUF, G, DW), jnp.int32),
            pltpu.SemaphoreType.DMA((NBUF,)),
            pltpu.SemaphoreType.DMA((NBUF,)),
        ],
    )(_sc_kernel_body)
    return kfn(table, idx_grouped)


BN = 1024
BP = 512


def _prep_body(x_ref, o_ref):
    bits = lax.bitcast_convert_type(x_ref[...], jnp.int32)
    rne = ((bits >> 16) & jnp.int32(1)) + jnp.int32(0x7FFF)
    bf = ((bits + rne) >> 16) & jnp.int32(0xFFFF)
    key = bf ^ (jnp.int32(0x7FFF) * ((bf >> 15) & jnp.int32(1)))
    words = key[:DW, :] | (key[DW:, :] << 16)
    o_ref[...] = words.T


def _post_body(w_ref, o_ref):
    w = w_ref[...]
    lo = w & jnp.int32(0xFFFF)
    hi = (w >> 16) & jnp.int32(0xFFFF)
    keys = jnp.concatenate([lo, hi], axis=1)
    ubf = keys ^ (jnp.int32(0x7FFF) * ((keys >> 15) & jnp.int32(1)))
    vals = lax.bitcast_convert_type(ubf << 16, jnp.float32)
    o_ref[...] = vals.T


@jax.jit
def _prep_tc(x2d):
    return pl.pallas_call(
        _prep_body,
        grid=(N_IN // BN,),
        in_specs=[pl.BlockSpec((D, BN), lambda i: (0, i))],
        out_specs=pl.BlockSpec((BN, DW), lambda i: (i, 0)),
        out_shape=jax.ShapeDtypeStruct((N_IN, DW), jnp.int32),
    )(x2d)


@jax.jit
def _post_tc(out_w):
    return pl.pallas_call(
        _post_body,
        grid=(N_OUT // BP,),
        in_specs=[pl.BlockSpec((BP, DW), lambda i: (i, 0))],
        out_specs=pl.BlockSpec((D, BP), lambda i: (0, i)),
        out_shape=jax.ShapeDtypeStruct((D, N_OUT), jnp.float32),
    )(out_w)


def kernel(x, idx):
    table = _prep_tc(x.reshape(D, N_IN))
    idx_grouped = idx.reshape(NW, NBLK, G * K)
    out_w = _max_pool_sc(table, idx_grouped)
    return _post_tc(out_w).reshape(B, C, N_OUT)


# speedup vs baseline: 1.3547x; 1.0555x over previous
import functools

import jax
import jax.numpy as jnp
from jax import lax
from jax.experimental import pallas as pl
from jax.experimental.pallas import tpu as pltpu
from jax.experimental.pallas import tpu_sc as plsc

B, C, N_IN = 4, 128, 32768
N_OUT, K = 8192, 16
D = B * C
DW = D // 2
L = 16

NC, NS = 2, 16
NW = NC * NS
ROWS_PER_W = N_OUT // NW
G = 8
NBLK = ROWS_PER_W // G
NBUF = 3
NITER = NBLK // NBUF + (1 if NBLK % NBUF else 0)


def _sc_kernel_body(table_hbm, idx_hbm, out_hbm, idx_v,
                    rows_bufs, out_bufs, gsems, osems):
    wid = lax.axis_index("s") * NC + lax.axis_index("c")
    base = wid * ROWS_PER_W

    pltpu.sync_copy(idx_hbm.at[wid], idx_v)

    for b in range(NBUF):
        pltpu.async_copy(table_hbm.at[idx_v.at[b]], rows_bufs.at[b],
                         gsems.at[b])

    def iter_body(i, carry):
        for b in range(NBUF):
            g = i * NBUF + b

            @pl.when(g < NBLK)
            def _(b=b, g=g):
                rows_v = rows_bufs.at[b]
                out_v = out_bufs.at[b]
                pltpu.make_async_copy(table_hbm.at[idx_v.at[g]], rows_v,
                                      gsems.at[b]).wait()
                @pl.when(i > 0)
                def _():
                    pltpu.make_async_copy(
                        out_v, out_hbm.at[pl.ds(base, G)], osems.at[b]).wait()

                for r in range(G):
                    def dg_body(dg, c2, r=r):
                        off = dg * L
                        va, vb = [], []
                        for j in range(K):
                            w = rows_v[r * K + j, pl.ds(off, L)]
                            va.append(w << 16)
                            vb.append(w)
                        for vals in (va, vb):
                            while len(vals) > 1:
                                vals[:] = [
                                    jnp.maximum(vals[2 * m], vals[2 * m + 1])
                                    for m in range(len(vals) // 2)]
                        out_v[r, pl.ds(off, L)] = (
                            ((va[0] >> 16) & jnp.int32(0xFFFF))
                            | (vb[0] & jnp.int32(-65536)))
                        return c2
                    lax.fori_loop(0, DW // L, dg_body, 0, unroll=1)

                @pl.when(g + NBUF < NBLK)
                def _():
                    pltpu.async_copy(table_hbm.at[idx_v.at[g + NBUF]],
                                     rows_v, gsems.at[b])
                pltpu.async_copy(out_v, out_hbm.at[pl.ds(base + g * G, G)],
                                 osems.at[b])
        return carry

    lax.fori_loop(0, NITER, iter_body, 0)

    for b in range(NBUF):
        pltpu.make_async_copy(out_bufs.at[b], out_hbm.at[pl.ds(base, G)],
                              osems.at[b]).wait()


@jax.jit
def _max_pool_sc(table, idx_grouped):
    mesh = plsc.VectorSubcoreMesh(core_axis_name="c", subcore_axis_name="s")
    kfn = functools.partial(
        pl.kernel,
        mesh=mesh,
        out_type=jax.ShapeDtypeStruct((N_OUT, DW), jnp.int32),
        scratch_types=[
            pltpu.VMEM((NBLK, G * K), jnp.int32),
            pltpu.VMEM((NBUF, G * K, DW), jnp.int32),
            pltpu.VMEM((NBUF, G, DW), jnp.int32),
            pltpu.SemaphoreType.DMA((NBUF,)),
            pltpu.SemaphoreType.DMA((NBUF,)),
        ],
    )(_sc_kernel_body)
    return kfn(table, idx_grouped)


BN = 2048
BP = 512


def _prep_body(x_ref, o_ref):
    bits = lax.bitcast_convert_type(x_ref[...], jnp.int32)
    rne = ((bits >> 16) & jnp.int32(1)) + jnp.int32(0x7FFF)
    bf = ((bits + rne) >> 16) & jnp.int32(0xFFFF)
    key = bf ^ (jnp.int32(0x7FFF) * ((bf >> 15) & jnp.int32(1)))
    words = key[:DW, :] | (key[DW:, :] << 16)
    o_ref[...] = words.T


def _post_body(w_ref, o_ref):
    w = w_ref[...]
    lo = w & jnp.int32(0xFFFF)
    hi = (w >> 16) & jnp.int32(0xFFFF)
    keys = jnp.concatenate([lo, hi], axis=1)
    ubf = keys ^ (jnp.int32(0x7FFF) * ((keys >> 15) & jnp.int32(1)))
    vals = lax.bitcast_convert_type(ubf << 16, jnp.float32)
    o_ref[...] = vals.T


@jax.jit
def _prep_tc(x2d):
    return pl.pallas_call(
        _prep_body,
        grid=(N_IN // BN,),
        in_specs=[pl.BlockSpec((D, BN), lambda i: (0, i))],
        out_specs=pl.BlockSpec((BN, DW), lambda i: (i, 0)),
        out_shape=jax.ShapeDtypeStruct((N_IN, DW), jnp.int32),
    )(x2d)


@jax.jit
def _post_tc(out_w):
    return pl.pallas_call(
        _post_body,
        grid=(N_OUT // BP,),
        in_specs=[pl.BlockSpec((BP, DW), lambda i: (i, 0))],
        out_specs=pl.BlockSpec((D, BP), lambda i: (0, i)),
        out_shape=jax.ShapeDtypeStruct((D, N_OUT), jnp.float32),
    )(out_w)


def kernel(x, idx):
    table = _prep_tc(x.reshape(D, N_IN))
    idx_grouped = idx.reshape(NW, NBLK, G * K)
    out_w = _max_pool_sc(table, idx_grouped)
    return _post_tc(out_w).reshape(B, C, N_OUT)


# speedup vs baseline: 1.3750x; 1.0150x over previous
import functools

import jax
import jax.numpy as jnp
from jax import lax
from jax.experimental import pallas as pl
from jax.experimental.pallas import tpu as pltpu
from jax.experimental.pallas import tpu_sc as plsc

B, C, N_IN = 4, 128, 32768
N_OUT, K = 8192, 16
D = B * C
DW = D // 2
L = 16

NC, NS = 2, 16
NW = NC * NS
ROWS_PER_W = N_OUT // NW
G = 8
NBLK = ROWS_PER_W // G
NBUF = 3
NITER = NBLK // NBUF + (1 if NBLK % NBUF else 0)


def _sc_kernel_body(table_hbm, idx_hbm, out_hbm, idx_v,
                    rows_bufs, out_bufs, gsems, osems):
    wid = lax.axis_index("s") * NC + lax.axis_index("c")
    base = wid * ROWS_PER_W

    pltpu.sync_copy(idx_hbm.at[wid], idx_v)

    for b in range(NBUF):
        pltpu.async_copy(table_hbm.at[idx_v.at[b]], rows_bufs.at[b],
                         gsems.at[b])

    def iter_body(i, carry):
        for b in range(NBUF):
            g = i * NBUF + b

            @pl.when(g < NBLK)
            def _(b=b, g=g):
                rows_v = rows_bufs.at[b]
                out_v = out_bufs.at[b]
                pltpu.make_async_copy(table_hbm.at[idx_v.at[g]], rows_v,
                                      gsems.at[b]).wait()
                @pl.when(i > 0)
                def _():
                    pltpu.make_async_copy(
                        out_v, out_hbm.at[pl.ds(base, G)], osems.at[b]).wait()

                for r in range(G):
                    def dg_body(dg, c2, r=r):
                        off = dg * L
                        va, vb = [], []
                        for j in range(K):
                            w = rows_v[r * K + j, pl.ds(off, L)]
                            va.append(w << 16)
                            vb.append(w)
                        for vals in (va, vb):
                            while len(vals) > 1:
                                vals[:] = [
                                    jnp.maximum(vals[2 * m], vals[2 * m + 1])
                                    for m in range(len(vals) // 2)]
                        out_v[r, pl.ds(off, L)] = (
                            ((va[0] >> 16) & jnp.int32(0xFFFF))
                            | (vb[0] & jnp.int32(-65536)))
                        return c2
                    lax.fori_loop(0, DW // L, dg_body, 0, unroll=1)

                @pl.when(g + NBUF < NBLK)
                def _():
                    pltpu.async_copy(table_hbm.at[idx_v.at[g + NBUF]],
                                     rows_v, gsems.at[b])
                pltpu.async_copy(out_v, out_hbm.at[pl.ds(base + g * G, G)],
                                 osems.at[b])
        return carry

    lax.fori_loop(0, NITER, iter_body, 0)

    for b in range(NBUF):
        pltpu.make_async_copy(out_bufs.at[b], out_hbm.at[pl.ds(base, G)],
                              osems.at[b]).wait()


@jax.jit
def _max_pool_sc(table, idx_grouped):
    mesh = plsc.VectorSubcoreMesh(core_axis_name="c", subcore_axis_name="s")
    kfn = functools.partial(
        pl.kernel,
        mesh=mesh,
        out_type=jax.ShapeDtypeStruct((N_OUT, DW), jnp.int32),
        scratch_types=[
            pltpu.VMEM((NBLK, G * K), jnp.int32),
            pltpu.VMEM((NBUF, G * K, DW), jnp.int32),
            pltpu.VMEM((NBUF, G, DW), jnp.int32),
            pltpu.SemaphoreType.DMA((NBUF,)),
            pltpu.SemaphoreType.DMA((NBUF,)),
        ],
    )(_sc_kernel_body)
    return kfn(table, idx_grouped)


BN = 4096
BP = 512


def _prep_body(x_ref, o_ref):
    bits = lax.bitcast_convert_type(x_ref[...], jnp.int32)
    rne = ((bits >> 16) & jnp.int32(1)) + jnp.int32(0x7FFF)
    bf = ((bits + rne) >> 16) & jnp.int32(0xFFFF)
    key = bf ^ (jnp.int32(0x7FFF) * ((bf >> 15) & jnp.int32(1)))
    words = key[:DW, :] | (key[DW:, :] << 16)
    o_ref[...] = words.T


def _post_body(w_ref, o_ref):
    w = w_ref[...]
    lo = w & jnp.int32(0xFFFF)
    hi = (w >> 16) & jnp.int32(0xFFFF)
    keys = jnp.concatenate([lo, hi], axis=1)
    ubf = keys ^ (jnp.int32(0x7FFF) * ((keys >> 15) & jnp.int32(1)))
    vals = lax.bitcast_convert_type(ubf << 16, jnp.float32)
    o_ref[...] = vals.T


@jax.jit
def _prep_tc(x2d):
    return pl.pallas_call(
        _prep_body,
        grid=(N_IN // BN,),
        in_specs=[pl.BlockSpec((D, BN), lambda i: (0, i))],
        out_specs=pl.BlockSpec((BN, DW), lambda i: (i, 0)),
        out_shape=jax.ShapeDtypeStruct((N_IN, DW), jnp.int32),
    )(x2d)


@jax.jit
def _post_tc(out_w):
    return pl.pallas_call(
        _post_body,
        grid=(N_OUT // BP,),
        in_specs=[pl.BlockSpec((BP, DW), lambda i: (i, 0))],
        out_specs=pl.BlockSpec((D, BP), lambda i: (0, i)),
        out_shape=jax.ShapeDtypeStruct((D, N_OUT), jnp.float32),
    )(out_w)


def kernel(x, idx):
    table = _prep_tc(x.reshape(D, N_IN))
    idx_grouped = idx.reshape(NW, NBLK, G * K)
    out_w = _max_pool_sc(table, idx_grouped)
    return _post_tc(out_w).reshape(B, C, N_OUT)


# speedup vs baseline: 1.4286x; 1.0390x over previous
import functools

import jax
import jax.numpy as jnp
from jax import lax
from jax.experimental import pallas as pl
from jax.experimental.pallas import tpu as pltpu
from jax.experimental.pallas import tpu_sc as plsc

B, C, N_IN = 4, 128, 32768
N_OUT, K = 8192, 16
D = B * C
DW = D // 2
L = 16

NC, NS = 2, 16
NW = NC * NS
ROWS_PER_W = N_OUT // NW
G = 8
NBLK = ROWS_PER_W // G
NBUF = 3
NITER = NBLK // NBUF + (1 if NBLK % NBUF else 0)


def _sc_kernel_body(table_hbm, idx_hbm, out_hbm, idx_v,
                    rows_bufs, out_bufs, gsems, osems):
    wid = lax.axis_index("s") * NC + lax.axis_index("c")
    base = wid * ROWS_PER_W

    pltpu.sync_copy(idx_hbm.at[wid], idx_v)

    for b in range(NBUF):
        pltpu.async_copy(table_hbm.at[idx_v.at[b]], rows_bufs.at[b],
                         gsems.at[b])

    def iter_body(i, carry):
        for b in range(NBUF):
            g = i * NBUF + b

            @pl.when(g < NBLK)
            def _(b=b, g=g):
                rows_v = rows_bufs.at[b]
                out_v = out_bufs.at[b]
                pltpu.make_async_copy(table_hbm.at[idx_v.at[g]], rows_v,
                                      gsems.at[b]).wait()
                @pl.when(i > 0)
                def _():
                    pltpu.make_async_copy(
                        out_v, out_hbm.at[pl.ds(base, G)], osems.at[b]).wait()

                for r in range(G):
                    def dg_body(dg, c2, r=r):
                        off = dg * L
                        va, vb = [], []
                        for j in range(K):
                            w = rows_v[r * K + j, pl.ds(off, L)]
                            va.append(w << 16)
                            vb.append(w)
                        for vals in (va, vb):
                            while len(vals) > 1:
                                vals[:] = [
                                    jnp.maximum(vals[2 * m], vals[2 * m + 1])
                                    for m in range(len(vals) // 2)]
                        out_v[r, pl.ds(off, L)] = (
                            ((va[0] >> 16) & jnp.int32(0xFFFF))
                            | (vb[0] & jnp.int32(-65536)))
                        return c2
                    lax.fori_loop(0, DW // L, dg_body, 0, unroll=1)

                @pl.when(g + NBUF < NBLK)
                def _():
                    pltpu.async_copy(table_hbm.at[idx_v.at[g + NBUF]],
                                     rows_v, gsems.at[b])
                pltpu.async_copy(out_v, out_hbm.at[pl.ds(base + g * G, G)],
                                 osems.at[b])
        return carry

    lax.fori_loop(0, NITER, iter_body, 0)

    for b in range(NBUF):
        pltpu.make_async_copy(out_bufs.at[b], out_hbm.at[pl.ds(base, G)],
                              osems.at[b]).wait()


@jax.jit
def _max_pool_sc(table, idx_grouped):
    mesh = plsc.VectorSubcoreMesh(core_axis_name="c", subcore_axis_name="s")
    kfn = functools.partial(
        pl.kernel,
        mesh=mesh,
        out_type=jax.ShapeDtypeStruct((N_OUT, DW), jnp.int32),
        scratch_types=[
            pltpu.VMEM((NBLK, G * K), jnp.int32),
            pltpu.VMEM((NBUF, G * K, DW), jnp.int32),
            pltpu.VMEM((NBUF, G, DW), jnp.int32),
            pltpu.SemaphoreType.DMA((NBUF,)),
            pltpu.SemaphoreType.DMA((NBUF,)),
        ],
    )(_sc_kernel_body)
    return kfn(table, idx_grouped)


BN = 4096
BP = 2048


def _prep_body(x_ref, o_ref):
    bits = lax.bitcast_convert_type(x_ref[...], jnp.int32)
    rne = ((bits >> 16) & jnp.int32(1)) + jnp.int32(0x7FFF)
    bf = ((bits + rne) >> 16) & jnp.int32(0xFFFF)
    key = bf ^ (jnp.int32(0x7FFF) * ((bf >> 15) & jnp.int32(1)))
    words = key[:DW, :] | (key[DW:, :] << 16)
    o_ref[...] = words.T


def _post_body(w_ref, o_ref):
    w = w_ref[...]
    lo = w & jnp.int32(0xFFFF)
    hi = (w >> 16) & jnp.int32(0xFFFF)
    keys = jnp.concatenate([lo, hi], axis=1)
    ubf = keys ^ (jnp.int32(0x7FFF) * ((keys >> 15) & jnp.int32(1)))
    vals = lax.bitcast_convert_type(ubf << 16, jnp.float32)
    o_ref[...] = vals.T


@jax.jit
def _prep_tc(x2d):
    return pl.pallas_call(
        _prep_body,
        grid=(N_IN // BN,),
        in_specs=[pl.BlockSpec((D, BN), lambda i: (0, i))],
        out_specs=pl.BlockSpec((BN, DW), lambda i: (i, 0)),
        out_shape=jax.ShapeDtypeStruct((N_IN, DW), jnp.int32),
    )(x2d)


@jax.jit
def _post_tc(out_w):
    return pl.pallas_call(
        _post_body,
        grid=(N_OUT // BP,),
        in_specs=[pl.BlockSpec((BP, DW), lambda i: (i, 0))],
        out_specs=pl.BlockSpec((D, BP), lambda i: (0, i)),
        out_shape=jax.ShapeDtypeStruct((D, N_OUT), jnp.float32),
    )(out_w)


def kernel(x, idx):
    table = _prep_tc(x.reshape(D, N_IN))
    idx_grouped = idx.reshape(NW, NBLK, G * K)
    out_w = _max_pool_sc(table, idx_grouped)
    return _post_tc(out_w).reshape(B, C, N_OUT)
